# baseline (speedup 1.0000x reference)
import dataclasses
import functools

import jax
import jax.numpy as jnp
from jax import lax
from jax.experimental import pallas as pl
from jax.experimental.pallas import tpu as pltpu
from jax.experimental.pallas import tpu_sc as plsc

N = 10000
D = 128
DH = 64
E = 320000
HID = 256

NC = 2
NS = 16
NW = NC * NS
EDGES_PER_TILE = E // NW
CH = 80
NCH = EDGES_PER_TILE // CH
PAD_N = 10240
ROWS_PER_TILE = PAD_N // NS


def _pack_rne(o32):
    b = lax.bitcast_convert_type(o32, jnp.int32)
    rne = b + 0x7FFF + lax.bitwise_and(lax.shift_right_logical(b, 16), 1)
    hw = lax.shift_right_logical(rne, 16)
    lo = hw[:, :DH]
    hi = hw[:, DH:]
    return lax.bitwise_or(lo, lax.shift_left(hi, 16))


def _sc_aggregate(h_il, src, dst, zeros):
    mesh = plsc.VectorSubcoreMesh(core_axis_name="c", subcore_axis_name="s")
    cp = pltpu.CompilerParams()
    if "needs_layout_passes" in pltpu.CompilerParams.__dataclass_fields__:
        cp = dataclasses.replace(cp, needs_layout_passes=False)
    if "use_tc_tiling_on_sc" in pltpu.CompilerParams.__dataclass_fields__:
        cp = dataclasses.replace(cp, use_tc_tiling_on_sc=False)

    @functools.partial(
        pl.kernel,
        mesh=mesh,
        compiler_params=cp,
        out_type=jax.ShapeDtypeStruct((NC, PAD_N, D), jnp.float32),
        scratch_types=(
            [pltpu.VMEM((CH,), jnp.int32) for _ in range(2)]
            + [pltpu.VMEM((CH,), jnp.int32) for _ in range(4)]
            + [pltpu.VMEM((CH, DH), jnp.int32) for _ in range(2)]
            + [pltpu.VMEM((CH, D), jnp.float32) for _ in range(2)]
            + [pltpu.VMEM_SHARED((PAD_N, D), jnp.float32)]
            + [pltpu.SemaphoreType.DMA for _ in range(10)]
        ),
    )
    def agg_kernel(h_hbm, src_hbm, dst_hbm, z_hbm, out_hbm,
                   src0, src1, dst0, dst1, dst2, dst3,
                   p0, p1, f0, f1, acc_sh,
                   iss0, iss1, isd0, isd1, isd2, isd3, gs0, gs1, ss0, ss1):
        cid = lax.axis_index("c")
        sid = lax.axis_index("s")
        base = (cid * NS + sid) * EDGES_PER_TILE
        r0 = sid * ROWS_PER_TILE

        srcs = [src0, src1]
        dsts = [dst0, dst1, dst2, dst3]
        rows16 = [p0, p1]
        rowsf = [f0, f1]
        isems = [iss0, iss1]
        isemd = [isd0, isd1, isd2, isd3]
        gsem = [gs0, gs1]
        ssem = [ss0, ss1]

        def idxs_start(j, x):
            pltpu.async_copy(src_hbm.at[pl.ds(base + j * CH, CH)], srcs[x], isems[x])

        def idxs_wait(j, x):
            pltpu.make_async_copy(src_hbm.at[pl.ds(base + j * CH, CH)], srcs[x], isems[x]).wait()

        def idxd_start(j, x):
            pltpu.async_copy(dst_hbm.at[pl.ds(base + j * CH, CH)], dsts[x], isemd[x])

        def idxd_wait(j, x):
            pltpu.make_async_copy(dst_hbm.at[pl.ds(base + j * CH, CH)], dsts[x], isemd[x]).wait()

        def gather_start(r):
            pltpu.async_copy(h_hbm.at[srcs[r]], rows16[r], gsem[r])

        def gather_wait(r):
            pltpu.make_async_copy(h_hbm.at[srcs[r]], rows16[r], gsem[r]).wait()

        def scatter_start(r, d):
            pltpu.async_copy(rowsf[r], acc_sh.at[dsts[d]], ssem[r], add=True)

        def scatter_wait(r, d):
            pltpu.make_async_copy(rowsf[r], acc_sh.at[dsts[d]], ssem[r]).wait()

        def unpack(r):
            src_ref = rows16[r]
            dst_ref = rowsf[r]

            @pl.loop(0, CH)
            def _(row):
                for m in range(DH // 16):
                    w = src_ref[row, pl.ds(16 * m, 16)]
                    lo = plsc.bitcast(lax.shift_left(w, 16), jnp.float32)
                    hi = plsc.bitcast(
                        lax.bitwise_and(w, jnp.int32(-65536)), jnp.float32)
                    dst_ref[row, pl.ds(16 * m, 16)] = lo
                    dst_ref[row, pl.ds(DH + 16 * m, 16)] = hi

        def emit(c, r, d, first=False, do_g=True, do_i=True):
            rn = (r + 1) % 2
            gather_wait(r)
            if do_g:
                idxs_wait(c + 1, rn)
                gather_start(rn)
            unpack(r)
            idxd_wait(c, d)
            scatter_start(r, d)
            if not first:
                scatter_wait(rn, (d + 3) % 4)
            if do_i:
                idxs_start(c + 2, r)
                idxd_start(c + 2, (d + 2) % 4)

        idxs_start(0, 0)
        idxd_start(0, 0)
        idxs_start(1, 1)
        idxd_start(1, 1)
        idxs_wait(0, 0)
        gather_start(0)
        pltpu.sync_copy(z_hbm.at[pl.ds(r0, ROWS_PER_TILE)],
                        acc_sh.at[pl.ds(r0, ROWS_PER_TILE)])
        plsc.subcore_barrier()

        emit(0, 0, 0, first=True)
        emit(1, 1, 1)

        @pl.loop(0, (NCH - 5) // 4)
        def _(i):
            c0 = 2 + 4 * i
            emit(c0, 0, 2)
            emit(c0 + 1, 1, 3)
            emit(c0 + 2, 0, 0)
            emit(c0 + 3, 1, 1)

        emit(NCH - 3, 0, 2)
        emit(NCH - 2, 1, 3, do_i=False)
        emit(NCH - 1, 0, 0, do_g=False, do_i=False)
        scatter_wait(0, 0)

        plsc.subcore_barrier()

        pltpu.sync_copy(acc_sh.at[pl.ds(r0, ROWS_PER_TILE)],
                        out_hbm.at[cid, pl.ds(r0, ROWS_PER_TILE)])

    return agg_kernel(h_il, src, dst, zeros)


def _tc_mlp(h, agg0, agg1, W1, b1, W2, b2, act, want_il):
    BN = 1000

    def mlp_kernel(h_ref, a0_ref, a1_ref, W1_ref, b1_ref, W2_ref, b2_ref,
                   o_ref, *maybe_il):
        z = h_ref[...] + a0_ref[...] + a1_ref[...]
        t = jnp.dot(z, W1_ref[...], preferred_element_type=jnp.float32)
        t = t + b1_ref[...]
        t = jnp.where(t > 0, t, 0.2 * t)
        o = jnp.dot(t, W2_ref[...], preferred_element_type=jnp.float32)
        o = o + b2_ref[...]
        if act:
            o = jnp.where(o > 0, o, 0.2 * o)
        o_ref[...] = o
        if want_il:
            maybe_il[0][...] = _pack_rne(o)

    out_shape = [jax.ShapeDtypeStruct((N, D), jnp.float32)]
    out_specs = [pl.BlockSpec((BN, D), lambda i: (i, 0))]
    if want_il:
        out_shape.append(jax.ShapeDtypeStruct((N, DH), jnp.int32))
        out_specs.append(pl.BlockSpec((BN, DH), lambda i: (i, 0)))

    res = pl.pallas_call(
        mlp_kernel,
        grid=(N // BN,),
        in_specs=[
            pl.BlockSpec((BN, D), lambda i: (i, 0)),
            pl.BlockSpec((BN, D), lambda i: (i, 0)),
            pl.BlockSpec((BN, D), lambda i: (i, 0)),
            pl.BlockSpec((D, HID), lambda i: (0, 0)),
            pl.BlockSpec((1, HID), lambda i: (0, 0)),
            pl.BlockSpec((HID, D), lambda i: (0, 0)),
            pl.BlockSpec((1, D), lambda i: (0, 0)),
        ],
        out_specs=out_specs,
        out_shape=out_shape,
    )(h, agg0, agg1, W1, b1.reshape(1, HID), W2, b2.reshape(1, D))
    return res if want_il else (res[0], None)


def kernel(x, edge_index,
           W1_0, b1_0, W2_0, b2_0,
           W1_1, b1_1, W2_1, b2_1,
           W1_2, b1_2, W2_2, b2_2):
    src = edge_index[0]
    dst = edge_index[1]
    zeros = jnp.zeros((PAD_N, D), jnp.float32)
    params = [(W1_0, b1_0, W2_0, b2_0),
              (W1_1, b1_1, W2_1, b2_1),
              (W1_2, b1_2, W2_2, b2_2)]
    h = x
    h_il = _pack_rne(x)
    for l in range(3):
        agg = _sc_aggregate(h_il, src, dst, zeros)
        h, h_il = _tc_mlp(h, agg[0], agg[1], *params[l],
                          act=(l < 2), want_il=(l < 2))
    return h


# speedup vs baseline: 1.8348x; 1.8348x over previous
import functools

import jax
import jax.numpy as jnp
from jax import lax
from jax.experimental import pallas as pl
from jax.experimental.pallas import tpu as pltpu
from jax.experimental.pallas import tpu_sc as plsc

N = 10000
D = 128
E = 320000
HID = 256

NC = 2
NS = 16
NW = NC * NS
EDGES_PER_TILE = E // NW
CH = 128
NCH = EDGES_PER_TILE // CH
TAIL = EDGES_PER_TILE - NCH * CH
PAD_N = 10240
ROWS_PER_TILE = PAD_N // NS


def _sc_aggregate(h, src, dst, zeros):
    mesh = plsc.VectorSubcoreMesh(core_axis_name="c", subcore_axis_name="s")

    @functools.partial(
        pl.kernel,
        mesh=mesh,
        out_type=jax.ShapeDtypeStruct((NC, PAD_N, D), jnp.float32),
        scratch_types=[
            pltpu.VMEM((CH,), jnp.int32),
            pltpu.VMEM((CH,), jnp.int32),
            pltpu.VMEM((CH,), jnp.int32),
            pltpu.VMEM((CH,), jnp.int32),
            pltpu.VMEM((CH, D), jnp.float32),
            pltpu.VMEM((CH, D), jnp.float32),
            pltpu.VMEM((TAIL,), jnp.int32),
            pltpu.VMEM((TAIL,), jnp.int32),
            pltpu.VMEM((TAIL, D), jnp.float32),
            pltpu.VMEM_SHARED((PAD_N, D), jnp.float32),
            pltpu.SemaphoreType.DMA,
            pltpu.SemaphoreType.DMA,
            pltpu.SemaphoreType.DMA,
            pltpu.SemaphoreType.DMA,
            pltpu.SemaphoreType.DMA,
        ],
    )
    def agg_kernel(h_hbm, src_hbm, dst_hbm, z_hbm, out_hbm,
                   src_a, src_b, dst_a, dst_b, rows_a, rows_b,
                   src_t, dst_t, rows_t, acc_sh,
                   sem_a, sem_b, sem_ia, sem_ib, sem_t):
        cid = lax.axis_index("c")
        sid = lax.axis_index("s")
        base = (cid * NS + sid) * EDGES_PER_TILE
        row0 = sid * ROWS_PER_TILE
        tail0 = base + NCH * CH

        def idx_start(j, sbuf, dbuf, sem):
            pltpu.async_copy(src_hbm.at[pl.ds(base + j * CH, CH)], sbuf, sem)
            pltpu.async_copy(dst_hbm.at[pl.ds(base + j * CH, CH)], dbuf, sem)

        def idx_wait(j, sbuf, dbuf, sem):
            pltpu.make_async_copy(src_hbm.at[pl.ds(base + j * CH, CH)], sbuf, sem).wait()
            pltpu.make_async_copy(dst_hbm.at[pl.ds(base + j * CH, CH)], dbuf, sem).wait()

        def gather_start(sbuf, buf, sem):
            pltpu.async_copy(h_hbm.at[sbuf], buf, sem)

        def gather_wait(sbuf, buf, sem):
            pltpu.make_async_copy(h_hbm.at[sbuf], buf, sem).wait()

        def scatter(buf, dbuf):
            pltpu.sync_copy(buf, acc_sh.at[dbuf], add=True)

        idx_start(0, src_a, dst_a, sem_ia)
        pltpu.async_copy(src_hbm.at[pl.ds(tail0, TAIL)], src_t, sem_t)
        pltpu.async_copy(dst_hbm.at[pl.ds(tail0, TAIL)], dst_t, sem_t)
        idx_wait(0, src_a, dst_a, sem_ia)
        gather_start(src_a, rows_a, sem_a)
        idx_start(1, src_b, dst_b, sem_ib)
        pltpu.make_async_copy(src_hbm.at[pl.ds(tail0, TAIL)], src_t, sem_t).wait()
        pltpu.make_async_copy(dst_hbm.at[pl.ds(tail0, TAIL)], dst_t, sem_t).wait()
        pltpu.async_copy(h_hbm.at[src_t], rows_t, sem_t)
        pltpu.sync_copy(z_hbm.at[pl.ds(row0, ROWS_PER_TILE)],
                        acc_sh.at[pl.ds(row0, ROWS_PER_TILE)])
        plsc.subcore_barrier()

        @pl.loop(0, NCH, step=2)
        def _(j):
            gather_wait(src_a, rows_a, sem_a)
            idx_wait(j + 1, src_b, dst_b, sem_ib)
            gather_start(src_b, rows_b, sem_b)
            scatter(rows_a, dst_a)

            @pl.when(j + 2 < NCH)
            def _():
                idx_start(j + 2, src_a, dst_a, sem_ia)

            gather_wait(src_b, rows_b, sem_b)

            @pl.when(j + 2 < NCH)
            def _():
                idx_wait(j + 2, src_a, dst_a, sem_ia)
                gather_start(src_a, rows_a, sem_a)

            scatter(rows_b, dst_b)

            @pl.when(j + 3 < NCH)
            def _():
                idx_start(j + 3, src_b, dst_b, sem_ib)

        pltpu.make_async_copy(h_hbm.at[src_t], rows_t, sem_t).wait()
        pltpu.sync_copy(rows_t, acc_sh.at[dst_t], add=True)

        plsc.subcore_barrier()

        pltpu.sync_copy(acc_sh.at[pl.ds(row0, ROWS_PER_TILE)],
                        out_hbm.at[cid, pl.ds(row0, ROWS_PER_TILE)])

    return agg_kernel(h, src, dst, zeros)


def _tc_mlp(h, agg0, agg1, W1, b1, W2, b2, act):
    BN = 1000

    def mlp_kernel(h_ref, a0_ref, a1_ref, W1_ref, b1_ref, W2_ref, b2_ref, o_ref):
        z = h_ref[...] + a0_ref[...] + a1_ref[...]
        t = jnp.dot(z, W1_ref[...], preferred_element_type=jnp.float32)
        t = t + b1_ref[...]
        t = jnp.where(t > 0, t, 0.2 * t)
        o = jnp.dot(t, W2_ref[...], preferred_element_type=jnp.float32)
        o = o + b2_ref[...]
        if act:
            o = jnp.where(o > 0, o, 0.2 * o)
        o_ref[...] = o

    return pl.pallas_call(
        mlp_kernel,
        grid=(N // BN,),
        in_specs=[
            pl.BlockSpec((BN, D), lambda i: (i, 0)),
            pl.BlockSpec((BN, D), lambda i: (i, 0)),
            pl.BlockSpec((BN, D), lambda i: (i, 0)),
            pl.BlockSpec((D, HID), lambda i: (0, 0)),
            pl.BlockSpec((1, HID), lambda i: (0, 0)),
            pl.BlockSpec((HID, D), lambda i: (0, 0)),
            pl.BlockSpec((1, D), lambda i: (0, 0)),
        ],
        out_specs=pl.BlockSpec((BN, D), lambda i: (i, 0)),
        out_shape=jax.ShapeDtypeStruct((N, D), jnp.float32),
    )(h, agg0, agg1, W1, b1.reshape(1, HID), W2, b2.reshape(1, D))


def kernel(x, edge_index,
           W1_0, b1_0, W2_0, b2_0,
           W1_1, b1_1, W2_1, b2_1,
           W1_2, b1_2, W2_2, b2_2):
    src = edge_index[0]
    dst = edge_index[1]
    zeros = jnp.zeros((PAD_N, D), jnp.float32)
    params = [(W1_0, b1_0, W2_0, b2_0),
              (W1_1, b1_1, W2_1, b2_1),
              (W1_2, b1_2, W2_2, b2_2)]
    h = x
    for l in range(3):
        agg = _sc_aggregate(h, src, dst, zeros)
        h = _tc_mlp(h, agg[0], agg[1], *params[l], act=(l < 2))
    return h


# speedup vs baseline: 1.8739x; 1.0213x over previous
import functools

import jax
import jax.numpy as jnp
from jax import lax
from jax.experimental import pallas as pl
from jax.experimental.pallas import tpu as pltpu
from jax.experimental.pallas import tpu_sc as plsc

N = 10000
D = 128
E = 320000
HID = 256

NC = 2
NS = 16
NW = NC * NS
EDGES_PER_TILE = E // NW
CH = 128
NCH = EDGES_PER_TILE // CH
TAIL = EDGES_PER_TILE - NCH * CH
PAD_N = 10240
ROWS_PER_TILE = PAD_N // NS


def _sc_aggregate(h, edge_index, zeros):
    mesh = plsc.VectorSubcoreMesh(core_axis_name="c", subcore_axis_name="s")

    @functools.partial(
        pl.kernel,
        mesh=mesh,
        out_type=jax.ShapeDtypeStruct((NC, PAD_N, D), jnp.float32),
        scratch_types=[
            pltpu.VMEM((2, CH), jnp.int32),
            pltpu.VMEM((2, CH), jnp.int32),
            pltpu.VMEM((CH, D), jnp.float32),
            pltpu.VMEM((CH, D), jnp.float32),
            pltpu.VMEM((2, TAIL), jnp.int32),
            pltpu.VMEM((TAIL, D), jnp.float32),
            pltpu.VMEM_SHARED((PAD_N, D), jnp.float32),
            pltpu.SemaphoreType.DMA,
            pltpu.SemaphoreType.DMA,
            pltpu.SemaphoreType.DMA,
            pltpu.SemaphoreType.DMA,
            pltpu.SemaphoreType.DMA,
        ],
    )
    def agg_kernel(h_hbm, edge_hbm, z_hbm, out_hbm,
                   idx_a, idx_b, rows_a, rows_b,
                   idx_t, rows_t, acc_sh,
                   sem_a, sem_b, sem_ia, sem_ib, sem_t):
        cid = lax.axis_index("c")
        sid = lax.axis_index("s")
        wid = cid * NS + sid
        row0 = sid * ROWS_PER_TILE
        tail0 = NCH * CH

        def idx_start(j, buf, sem):
            pltpu.async_copy(edge_hbm.at[:, wid, pl.ds(j * CH, CH)], buf, sem)

        def idx_wait(j, buf, sem):
            pltpu.make_async_copy(edge_hbm.at[:, wid, pl.ds(j * CH, CH)], buf, sem).wait()

        def gather_start(buf, rbuf, sem):
            pltpu.async_copy(h_hbm.at[buf.at[0]], rbuf, sem)

        def gather_wait(buf, rbuf, sem):
            pltpu.make_async_copy(h_hbm.at[buf.at[0]], rbuf, sem).wait()

        def scatter(rbuf, buf):
            pltpu.sync_copy(rbuf, acc_sh.at[buf.at[1]], add=True)

        idx_start(0, idx_a, sem_ia)
        pltpu.async_copy(edge_hbm.at[:, wid, pl.ds(tail0, TAIL)], idx_t, sem_t)
        idx_wait(0, idx_a, sem_ia)
        gather_start(idx_a, rows_a, sem_a)
        idx_start(1, idx_b, sem_ib)
        pltpu.make_async_copy(edge_hbm.at[:, wid, pl.ds(tail0, TAIL)], idx_t, sem_t).wait()
        pltpu.async_copy(h_hbm.at[idx_t.at[0]], rows_t, sem_t)
        pltpu.sync_copy(z_hbm.at[pl.ds(row0, ROWS_PER_TILE)],
                        acc_sh.at[pl.ds(row0, ROWS_PER_TILE)])
        plsc.subcore_barrier()

        @pl.loop(0, NCH - 2, step=2)
        def _(j):
            gather_wait(idx_a, rows_a, sem_a)
            idx_wait(j + 1, idx_b, sem_ib)
            gather_start(idx_b, rows_b, sem_b)
            scatter(rows_a, idx_a)
            idx_start(j + 2, idx_a, sem_ia)
            gather_wait(idx_b, rows_b, sem_b)
            idx_wait(j + 2, idx_a, sem_ia)
            gather_start(idx_a, rows_a, sem_a)
            scatter(rows_b, idx_b)
            idx_start(j + 3, idx_b, sem_ib)

        gather_wait(idx_a, rows_a, sem_a)
        idx_wait(NCH - 1, idx_b, sem_ib)
        gather_start(idx_b, rows_b, sem_b)
        scatter(rows_a, idx_a)
        gather_wait(idx_b, rows_b, sem_b)
        scatter(rows_b, idx_b)

        pltpu.make_async_copy(h_hbm.at[idx_t.at[0]], rows_t, sem_t).wait()
        pltpu.sync_copy(rows_t, acc_sh.at[idx_t.at[1]], add=True)

        plsc.subcore_barrier()

        pltpu.sync_copy(acc_sh.at[pl.ds(row0, ROWS_PER_TILE)],
                        out_hbm.at[cid, pl.ds(row0, ROWS_PER_TILE)])

    return agg_kernel(h, edge_index.reshape(2, NW, EDGES_PER_TILE), zeros)


def _tc_mlp(h, agg0, agg1, W1, b1, W2, b2, act):
    BN = 1000

    def mlp_kernel(h_ref, a0_ref, a1_ref, W1_ref, b1_ref, W2_ref, b2_ref, o_ref):
        z = h_ref[...] + a0_ref[...] + a1_ref[...]
        t = jnp.dot(z, W1_ref[...], preferred_element_type=jnp.float32)
        t = t + b1_ref[...]
        t = jnp.where(t > 0, t, 0.2 * t)
        o = jnp.dot(t, W2_ref[...], preferred_element_type=jnp.float32)
        o = o + b2_ref[...]
        if act:
            o = jnp.where(o > 0, o, 0.2 * o)
        o_ref[...] = o

    return pl.pallas_call(
        mlp_kernel,
        grid=(N // BN,),
        in_specs=[
            pl.BlockSpec((BN, D), lambda i: (i, 0)),
            pl.BlockSpec((BN, D), lambda i: (i, 0)),
            pl.BlockSpec((BN, D), lambda i: (i, 0)),
            pl.BlockSpec((D, HID), lambda i: (0, 0)),
            pl.BlockSpec((1, HID), lambda i: (0, 0)),
            pl.BlockSpec((HID, D), lambda i: (0, 0)),
            pl.BlockSpec((1, D), lambda i: (0, 0)),
        ],
        out_specs=pl.BlockSpec((BN, D), lambda i: (i, 0)),
        out_shape=jax.ShapeDtypeStruct((N, D), jnp.float32),
    )(h, agg0, agg1, W1, b1.reshape(1, HID), W2, b2.reshape(1, D))


def kernel(x, edge_index,
           W1_0, b1_0, W2_0, b2_0,
           W1_1, b1_1, W2_1, b2_1,
           W1_2, b1_2, W2_2, b2_2):
    zeros = jnp.zeros((PAD_N, D), jnp.float32)
    params = [(W1_0, b1_0, W2_0, b2_0),
              (W1_1, b1_1, W2_1, b2_1),
              (W1_2, b1_2, W2_2, b2_2)]
    h = x
    for l in range(3):
        agg = _sc_aggregate(h, edge_index, zeros)
        h = _tc_mlp(h, agg[0], agg[1], *params[l], act=(l < 2))
    return h
